# baseline (speedup 1.0000x reference)
import functools
import jax
import jax.numpy as jnp
from jax import lax
from jax.experimental import pallas as pl
from jax.experimental.pallas import tpu as pltpu
from jax.experimental.pallas import tpu_sc as plsc

NC = 2
NS = 16
NT = NC * NS
CHUNK = 2000

_MESH = dict(core_axis_name="c", subcore_axis_name="s", num_cores=NC,
             num_subcores=NS)


def _round_up(a, m):
    return (a + m - 1) // m * m


IDXD = 10
ZR = 392


def _gs_kernel(npad, e_pad):
    F = 8
    per_tile = e_pad // NT
    n_chunks = per_tile // CHUNK
    n_blocks = n_chunks // IDXD
    slice_sz = npad // NS

    scratch = (
        [pltpu.VMEM((CHUNK,), jnp.int32) for _ in range(IDXD)] +
        [pltpu.VMEM((CHUNK,), jnp.int32) for _ in range(IDXD)] +
        [pltpu.VMEM((CHUNK, F), jnp.float32) for _ in range(2)] +
        [pltpu.VMEM_SHARED((npad, F), jnp.float32),
         pltpu.SemaphoreType.DMA, pltpu.SemaphoreType.DMA]
    )

    @functools.partial(
        pl.kernel,
        out_type=jax.ShapeDtypeStruct((NC, npad, F), jnp.float32),
        mesh=plsc.VectorSubcoreMesh(**_MESH),
        scratch_types=scratch,
        compiler_params=pltpu.CompilerParams(use_tc_tiling_on_sc=False),
    )
    def k(src_hbm, dst_hbm, table_hbm, zeros_hbm, out_hbm, *refs):
        sidx = refs[:IDXD]
        didx = refs[IDXD:2 * IDXD]
        rows = refs[2 * IDXD:2 * IDXD + 2]
        acc_sh = refs[2 * IDXD + 2]
        gsem = refs[2 * IDXD + 3:2 * IDXD + 5]
        c = lax.axis_index("c")
        s = lax.axis_index("s")
        sl = pl.ds(s * slice_sz, slice_sz)
        for r in range(slice_sz // ZR):
            pltpu.sync_copy(zeros_hbm,
                            acc_sh.at[pl.ds(s * slice_sz + r * ZR, ZR)])
        plsc.subcore_barrier()

        base = (c * NS + s) * per_tile

        def gather_start(j, b):
            pltpu.async_copy(table_hbm.at[sidx[j]], rows[b], gsem[b])

        def gather_wait(j, b):
            pltpu.make_async_copy(table_hbm.at[sidx[j]], rows[b],
                                  gsem[b]).wait()

        def blk(blk_i, _):
            eoff = base + blk_i * IDXD * CHUNK
            for j in range(IDXD):
                pltpu.sync_copy(dst_hbm.at[pl.ds(eoff + j * CHUNK, CHUNK)],
                                didx[j])
                pltpu.sync_copy(src_hbm.at[pl.ds(eoff + j * CHUNK, CHUNK)],
                                sidx[j])
            gather_start(0, 0)
            for j in range(IDXD):
                b = j & 1
                gather_wait(j, b)
                if j + 1 < IDXD:
                    gather_start(j + 1, 1 - b)
                pltpu.sync_copy(rows[b], acc_sh.at[didx[j]], add=True)
            return 0

        lax.fori_loop(0, n_blocks, blk, 0)
        plsc.subcore_barrier()
        pltpu.sync_copy(acc_sh.at[sl], out_hbm.at[c].at[sl])

    return k


def _deg_kernel(npad, e_pad):
    F = 8
    per_tile = e_pad // NT
    n_chunks = per_tile // CHUNK
    n_blocks = n_chunks // IDXD
    slice_sz = npad // NS

    xsl = npad // NT

    scratch = (
        [pltpu.VMEM((CHUNK,), jnp.int32) for _ in range(IDXD)] +
        [pltpu.VMEM((CHUNK, F), jnp.float32),
         pltpu.VMEM((xsl * F,), jnp.float32),
         pltpu.VMEM((xsl * 2,), jnp.float32),
         pltpu.VMEM_SHARED((npad, F), jnp.float32)]
    )

    @functools.partial(
        pl.kernel,
        out_type=(jax.ShapeDtypeStruct((NC, npad, F), jnp.float32),
                  jax.ShapeDtypeStruct((npad * F,), jnp.float32)),
        mesh=plsc.VectorSubcoreMesh(**_MESH),
        scratch_types=scratch,
        compiler_params=pltpu.CompilerParams(use_tc_tiling_on_sc=False,
                                             needs_layout_passes=False),
    )
    def k(dst_hbm, ones_hbm, zeros_hbm, zeros1_hbm, x2_hbm, out_hbm,
          xtab_hbm, *refs):
        didx = refs[:IDXD]
        ones_v = refs[IDXD]
        xv = refs[IDXD + 1]
        xin = refs[IDXD + 2]
        acc_sh = refs[IDXD + 3]
        c = lax.axis_index("c")
        s = lax.axis_index("s")
        sl = pl.ds(s * slice_sz, slice_sz)
        for r in range(slice_sz // ZR):
            pltpu.sync_copy(zeros_hbm,
                            acc_sh.at[pl.ds(s * slice_sz + r * ZR, ZR)])
        pltpu.sync_copy(ones_hbm, ones_v)

        wid = c * NS + s
        zchunk = ZR * 16
        for r in range(xsl * F // zchunk):
            pltpu.sync_copy(zeros1_hbm, xv.at[pl.ds(r * zchunk, zchunk)])
        pltpu.sync_copy(x2_hbm.at[pl.ds(wid * xsl * 2, xsl * 2)], xin)
        lane = jax.lax.iota(jnp.int32, 16)

        def xstep(i, _):
            p = i * 16 + lane
            vals = xin[pl.ds(i * 16, 16)]
            tgt = p * 4 - 3 * (p & 1)
            plsc.store_scatter(xv, [tgt], vals)
            return 0

        lax.fori_loop(0, xsl * 2 // 16, xstep, 0)
        pltpu.sync_copy(xv, xtab_hbm.at[pl.ds(wid * xsl * F, xsl * F)])

        plsc.subcore_barrier()
        base = (c * NS + s) * per_tile

        def blk(blk_i, _):
            eoff = base + blk_i * IDXD * CHUNK
            for j in range(IDXD):
                pltpu.sync_copy(dst_hbm.at[pl.ds(eoff + j * CHUNK, CHUNK)],
                                didx[j])
            for j in range(IDXD):
                pltpu.sync_copy(ones_v, acc_sh.at[didx[j]], add=True)
            return 0

        lax.fori_loop(0, n_blocks, blk, 0)
        plsc.subcore_barrier()
        pltpu.sync_copy(acc_sh.at[sl], out_hbm.at[c].at[sl])

    return k


def _prep_call(npad, deg_p, x_flat):
    def body(degp, xf, disf, xnf):
        deg = degp[0] + degp[1] + 1.0
        d = lax.rsqrt(deg)
        disf[...] = d
        xnf[...] = xf[...] * d

    G = npad // 16
    R = 784
    g = G // R
    return pl.pallas_call(
        body,
        grid=(g,),
        in_specs=[
            pl.BlockSpec((NC, R, 128), lambda i: (0, i, 0)),
            pl.BlockSpec((R, 128), lambda i: (i, 0)),
        ],
        out_specs=(
            pl.BlockSpec((R, 128), lambda i: (i, 0)),
            pl.BlockSpec((R, 128), lambda i: (i, 0)),
        ),
        out_shape=(
            jax.ShapeDtypeStruct((G, 128), jnp.float32),
            jax.ShapeDtypeStruct((G, 128), jnp.float32),
        ),
    )(deg_p, x_flat)


def _mid_call(npad, t1_p, xnf, disf, W1big, b1big, W2big):
    G = npad // 16
    R = 784
    g = G // R

    def body(tp, xnb, disb, w1, bb1, w2, gnf):
        agg = (tp[0] + tp[1] + xnb[...]) * disb[...]
        h1 = jnp.dot(agg, w1[...], preferred_element_type=jnp.float32)
        h1 = jnp.maximum(h1 + bb1[...], 0.0)
        gnf[...] = jnp.dot(h1, w2[...],
                           preferred_element_type=jnp.float32) * disb[...]

    return pl.pallas_call(
        body,
        grid=(g,),
        in_specs=[
            pl.BlockSpec((NC, R, 128), lambda i: (0, i, 0)),
            pl.BlockSpec((R, 128), lambda i: (i, 0)),
            pl.BlockSpec((R, 128), lambda i: (i, 0)),
            pl.BlockSpec((128, 256), lambda i: (0, 0)),
            pl.BlockSpec((1, 256), lambda i: (0, 0)),
            pl.BlockSpec((256, 128), lambda i: (0, 0)),
        ],
        out_specs=pl.BlockSpec((R, 128), lambda i: (i, 0)),
        out_shape=jax.ShapeDtypeStruct((G, 128), jnp.float32),
    )(t1_p, xnf, disf, W1big, b1big.reshape(1, 256), W2big)


def _post_call(npad, t2_p, gnf, disf):
    G = npad // 16
    R = 784
    g = G // R

    def body(tp, gnb, disb, aggf):
        aggf[...] = (tp[0] + tp[1] + gnb[...]) * disb[...]

    return pl.pallas_call(
        body,
        grid=(g,),
        in_specs=[
            pl.BlockSpec((NC, R, 128), lambda i: (0, i, 0)),
            pl.BlockSpec((R, 128), lambda i: (i, 0)),
            pl.BlockSpec((R, 128), lambda i: (i, 0)),
        ],
        out_specs=pl.BlockSpec((R, 128), lambda i: (i, 0)),
        out_shape=jax.ShapeDtypeStruct((G, 128), jnp.float32),
    )(t2_p, gnf, disf)


def _final_call(n, agg2, fcW, b2, fcb):
    R = 4000
    g = n // R

    def body(ab, w, bb2, bfc, out):
        cvec = jnp.dot(bb2[...], w[...],
                       preferred_element_type=jnp.float32) + bfc[...]
        out[...] = jnp.dot(ab[...], w[...],
                           preferred_element_type=jnp.float32) + cvec

    return pl.pallas_call(
        body,
        grid=(g,),
        in_specs=[
            pl.BlockSpec((R, 8), lambda i: (i, 0)),
            pl.BlockSpec((8, 256), lambda i: (0, 0)),
            pl.BlockSpec((1, 8), lambda i: (0, 0)),
            pl.BlockSpec((1, 256), lambda i: (0, 0)),
        ],
        out_specs=pl.BlockSpec((R, 256), lambda i: (i, 0)),
        out_shape=jax.ShapeDtypeStruct((n, 256), jnp.float32),
    )(agg2, fcW, b2.reshape(1, 8), fcb.reshape(1, 256))


@jax.jit
def kernel(x, edge_index, W1, b1, W2, b2, fcW, fcb):
    n = x.shape[0]
    e = edge_index.shape[1]
    npad = _round_up(n + 1, 2048)
    e_pad = _round_up(e, NT * CHUNK * IDXD)

    src = edge_index[0]
    dst = edge_index[1]
    if e_pad != e:
        pad = jnp.full((e_pad - e,), n, dtype=jnp.int32)
        src = jnp.concatenate([src, pad])
        dst = jnp.concatenate([dst, pad])

    G = npad // 16
    W1p = jnp.zeros((8, 16), jnp.float32).at[:2].set(W1)
    W1big = jnp.kron(jnp.eye(16, dtype=jnp.float32), W1p)
    W2big = jnp.kron(jnp.eye(16, dtype=jnp.float32), W2)
    b1big = jnp.tile(b1, 16)
    zeros_s = jnp.zeros((ZR, 8), jnp.float32)
    zeros_1 = jnp.zeros((ZR * 16,), jnp.float32)
    ones_c = jnp.ones((CHUNK, 8), jnp.float32)
    x2_pad = jnp.pad(x, ((0, npad - n), (0, 0))).reshape(npad * 2)

    deg_p, xtab = _deg_kernel(npad, e_pad)(dst, ones_c, zeros_s, zeros_1,
                                           x2_pad)

    disf, xnf = _prep_call(npad, deg_p.reshape(NC, G, 128),
                           xtab.reshape(G, 128))

    t1_p = _gs_kernel(npad, e_pad)(src, dst, xnf.reshape(npad, 8), zeros_s)

    gnf = _mid_call(npad, t1_p.reshape(NC, G, 128), xnf, disf,
                    W1big, b1big, W2big)

    t2_p = _gs_kernel(npad, e_pad)(src, dst, gnf.reshape(npad, 8), zeros_s)

    agg2f = _post_call(npad, t2_p.reshape(NC, G, 128), gnf, disf)
    return _final_call(n, agg2f.reshape(npad, 8), fcW, b2, fcb)


# speedup vs baseline: 1.3587x; 1.3587x over previous
import functools
import jax
import jax.numpy as jnp
from jax import lax
from jax.experimental import pallas as pl
from jax.experimental.pallas import tpu as pltpu
from jax.experimental.pallas import tpu_sc as plsc

NC = 2
NS = 16
NT = NC * NS
CHUNK = 4000

_MESH = dict(core_axis_name="c", subcore_axis_name="s", num_cores=NC,
             num_subcores=NS)


def _round_up(a, m):
    return (a + m - 1) // m * m


ZR = 392


def _gs_kernel(npad, e_pad):
    F = 8
    per_tile = e_pad // NT
    n_chunks = per_tile // CHUNK
    slice_sz = npad // NS

    scratch = (
        [pltpu.VMEM((CHUNK,), jnp.int32) for _ in range(2)] +
        [pltpu.VMEM((CHUNK,), jnp.int32) for _ in range(2)] +
        [pltpu.VMEM((CHUNK, F), jnp.float32) for _ in range(2)] +
        [pltpu.VMEM_SHARED((npad, F), jnp.float32),
         pltpu.SemaphoreType.DMA, pltpu.SemaphoreType.DMA,
         pltpu.SemaphoreType.DMA, pltpu.SemaphoreType.DMA]
    )

    @functools.partial(
        pl.kernel,
        out_type=jax.ShapeDtypeStruct((NC, npad, F), jnp.float32),
        mesh=plsc.VectorSubcoreMesh(**_MESH),
        scratch_types=scratch,
        compiler_params=pltpu.CompilerParams(use_tc_tiling_on_sc=False),
    )
    def k(src_hbm, dst_hbm, table_hbm, zeros_hbm, out_hbm, *refs):
        sidx = refs[0:2]
        didx = refs[2:4]
        rows = refs[4:6]
        acc_sh = refs[6]
        gsem = refs[7:9]
        isem = refs[9:11]
        c = lax.axis_index("c")
        s = lax.axis_index("s")
        sl = pl.ds(s * slice_sz, slice_sz)
        for r in range(slice_sz // ZR):
            pltpu.sync_copy(zeros_hbm,
                            acc_sh.at[pl.ds(s * slice_sz + r * ZR, ZR)])
        plsc.subcore_barrier()

        base = (c * NS + s) * per_tile

        def off(kk):
            return pl.ds(base + kk * CHUNK, CHUNK)

        def idx_start(kk, b):
            pltpu.async_copy(dst_hbm.at[off(kk)], didx[b], isem[b])
            pltpu.async_copy(src_hbm.at[off(kk)], sidx[b], isem[b])

        def idx_wait(kk, b):
            pltpu.make_async_copy(dst_hbm.at[off(kk)], didx[b], isem[b]).wait()
            pltpu.make_async_copy(src_hbm.at[off(kk)], sidx[b],
                                  isem[b]).wait()

        def gather_start(b):
            pltpu.async_copy(table_hbm.at[sidx[b]], rows[b], gsem[b])

        def gather_wait(b):
            pltpu.make_async_copy(table_hbm.at[sidx[b]], rows[b],
                                  gsem[b]).wait()

        idx_start(0, 0)
        idx_wait(0, 0)
        gather_start(0)
        if n_chunks > 1:
            idx_start(1, 1)
        for kk in range(n_chunks):
            b = kk & 1
            nb = 1 - b
            gather_wait(b)
            if kk + 1 < n_chunks:
                idx_wait(kk + 1, nb)
                gather_start(nb)
            pltpu.sync_copy(rows[b], acc_sh.at[didx[b]], add=True)
            if kk + 2 < n_chunks:
                idx_start(kk + 2, b)

        plsc.subcore_barrier()
        pltpu.sync_copy(acc_sh.at[sl], out_hbm.at[c].at[sl])

    return k


def _deg_kernel(npad, e_pad):
    F = 8
    per_tile = e_pad // NT
    n_chunks = per_tile // CHUNK
    slice_sz = npad // NS

    xsl = npad // NT

    scratch = (
        [pltpu.VMEM((CHUNK,), jnp.int32) for _ in range(2)] +
        [pltpu.VMEM((CHUNK, F), jnp.float32),
         pltpu.VMEM((xsl * F,), jnp.float32),
         pltpu.VMEM((xsl * 2,), jnp.float32),
         pltpu.VMEM_SHARED((npad, F), jnp.float32),
         pltpu.SemaphoreType.DMA, pltpu.SemaphoreType.DMA]
    )

    @functools.partial(
        pl.kernel,
        out_type=(jax.ShapeDtypeStruct((NC, npad, F), jnp.float32),
                  jax.ShapeDtypeStruct((npad * F,), jnp.float32)),
        mesh=plsc.VectorSubcoreMesh(**_MESH),
        scratch_types=scratch,
        compiler_params=pltpu.CompilerParams(use_tc_tiling_on_sc=False,
                                             needs_layout_passes=False),
    )
    def k(dst_hbm, ones_hbm, zeros_hbm, zeros1_hbm, x2_hbm, out_hbm,
          xtab_hbm, *refs):
        didx = refs[0:2]
        ones_v = refs[2]
        xv = refs[3]
        xin = refs[4]
        acc_sh = refs[5]
        isem = refs[6:8]
        c = lax.axis_index("c")
        s = lax.axis_index("s")
        sl = pl.ds(s * slice_sz, slice_sz)
        for r in range(slice_sz // ZR):
            pltpu.sync_copy(zeros_hbm,
                            acc_sh.at[pl.ds(s * slice_sz + r * ZR, ZR)])
        pltpu.sync_copy(ones_hbm, ones_v)

        wid = c * NS + s
        zchunk = ZR * 16
        for r in range(xsl * F // zchunk):
            pltpu.sync_copy(zeros1_hbm, xv.at[pl.ds(r * zchunk, zchunk)])
        pltpu.sync_copy(x2_hbm.at[pl.ds(wid * xsl * 2, xsl * 2)], xin)
        lane = jax.lax.iota(jnp.int32, 16)

        def xstep(i, _):
            p = i * 16 + lane
            vals = xin[pl.ds(i * 16, 16)]
            tgt = p * 4 - 3 * (p & 1)
            plsc.store_scatter(xv, [tgt], vals)
            return 0

        lax.fori_loop(0, xsl * 2 // 16, xstep, 0)
        pltpu.sync_copy(xv, xtab_hbm.at[pl.ds(wid * xsl * F, xsl * F)])

        plsc.subcore_barrier()
        base = (c * NS + s) * per_tile

        def off(kk):
            return pl.ds(base + kk * CHUNK, CHUNK)

        def idx_start(kk, b):
            pltpu.async_copy(dst_hbm.at[off(kk)], didx[b], isem[b])

        def idx_wait(kk, b):
            pltpu.make_async_copy(dst_hbm.at[off(kk)], didx[b], isem[b]).wait()

        idx_start(0, 0)
        if n_chunks > 1:
            idx_start(1, 1)
        for kk in range(n_chunks):
            b = kk & 1
            idx_wait(kk, b)
            pltpu.sync_copy(ones_v, acc_sh.at[didx[b]], add=True)
            if kk + 2 < n_chunks:
                idx_start(kk + 2, b)

        plsc.subcore_barrier()
        pltpu.sync_copy(acc_sh.at[sl], out_hbm.at[c].at[sl])

    return k


def _prep_call(npad, deg_p, x_flat):
    def body(degp, xf, disf, xnf):
        deg = degp[0] + degp[1] + 1.0
        d = lax.rsqrt(deg)
        disf[...] = d
        xnf[...] = xf[...] * d

    G = npad // 16
    R = 784
    g = G // R
    return pl.pallas_call(
        body,
        grid=(g,),
        in_specs=[
            pl.BlockSpec((NC, R, 128), lambda i: (0, i, 0)),
            pl.BlockSpec((R, 128), lambda i: (i, 0)),
        ],
        out_specs=(
            pl.BlockSpec((R, 128), lambda i: (i, 0)),
            pl.BlockSpec((R, 128), lambda i: (i, 0)),
        ),
        out_shape=(
            jax.ShapeDtypeStruct((G, 128), jnp.float32),
            jax.ShapeDtypeStruct((G, 128), jnp.float32),
        ),
    )(deg_p, x_flat)


def _mid_call(npad, t1_p, xnf, disf, W1big, b1big, W2big):
    G = npad // 16
    R = 784
    g = G // R

    def body(tp, xnb, disb, w1, bb1, w2, gnf):
        agg = (tp[0] + tp[1] + xnb[...]) * disb[...]
        h1 = jnp.dot(agg, w1[...], preferred_element_type=jnp.float32)
        h1 = jnp.maximum(h1 + bb1[...], 0.0)
        gnf[...] = jnp.dot(h1, w2[...],
                           preferred_element_type=jnp.float32) * disb[...]

    return pl.pallas_call(
        body,
        grid=(g,),
        in_specs=[
            pl.BlockSpec((NC, R, 128), lambda i: (0, i, 0)),
            pl.BlockSpec((R, 128), lambda i: (i, 0)),
            pl.BlockSpec((R, 128), lambda i: (i, 0)),
            pl.BlockSpec((128, 256), lambda i: (0, 0)),
            pl.BlockSpec((1, 256), lambda i: (0, 0)),
            pl.BlockSpec((256, 128), lambda i: (0, 0)),
        ],
        out_specs=pl.BlockSpec((R, 128), lambda i: (i, 0)),
        out_shape=jax.ShapeDtypeStruct((G, 128), jnp.float32),
    )(t1_p, xnf, disf, W1big, b1big.reshape(1, 256), W2big)


def _post_call(npad, t2_p, gnf, disf):
    G = npad // 16
    R = 784
    g = G // R

    def body(tp, gnb, disb, aggf):
        aggf[...] = (tp[0] + tp[1] + gnb[...]) * disb[...]

    return pl.pallas_call(
        body,
        grid=(g,),
        in_specs=[
            pl.BlockSpec((NC, R, 128), lambda i: (0, i, 0)),
            pl.BlockSpec((R, 128), lambda i: (i, 0)),
            pl.BlockSpec((R, 128), lambda i: (i, 0)),
        ],
        out_specs=pl.BlockSpec((R, 128), lambda i: (i, 0)),
        out_shape=jax.ShapeDtypeStruct((G, 128), jnp.float32),
    )(t2_p, gnf, disf)


def _final_call(n, agg2, fcW, b2, fcb):
    R = 4000
    g = n // R

    def body(ab, w, bb2, bfc, out):
        cvec = jnp.dot(bb2[...], w[...],
                       preferred_element_type=jnp.float32) + bfc[...]
        out[...] = jnp.dot(ab[...], w[...],
                           preferred_element_type=jnp.float32) + cvec

    return pl.pallas_call(
        body,
        grid=(g,),
        in_specs=[
            pl.BlockSpec((R, 8), lambda i: (i, 0)),
            pl.BlockSpec((8, 256), lambda i: (0, 0)),
            pl.BlockSpec((1, 8), lambda i: (0, 0)),
            pl.BlockSpec((1, 256), lambda i: (0, 0)),
        ],
        out_specs=pl.BlockSpec((R, 256), lambda i: (i, 0)),
        out_shape=jax.ShapeDtypeStruct((n, 256), jnp.float32),
    )(agg2, fcW, b2.reshape(1, 8), fcb.reshape(1, 256))


@jax.jit
def kernel(x, edge_index, W1, b1, W2, b2, fcW, fcb):
    n = x.shape[0]
    e = edge_index.shape[1]
    npad = _round_up(n + 1, 2048)
    e_pad = _round_up(e, NT * CHUNK)

    src = edge_index[0]
    dst = edge_index[1]
    if e_pad != e:
        pad = jnp.full((e_pad - e,), n, dtype=jnp.int32)
        src = jnp.concatenate([src, pad])
        dst = jnp.concatenate([dst, pad])

    G = npad // 16
    W1p = jnp.zeros((8, 16), jnp.float32).at[:2].set(W1)
    W1big = jnp.kron(jnp.eye(16, dtype=jnp.float32), W1p)
    W2big = jnp.kron(jnp.eye(16, dtype=jnp.float32), W2)
    b1big = jnp.tile(b1, 16)
    zeros_s = jnp.zeros((ZR, 8), jnp.float32)
    zeros_1 = jnp.zeros((ZR * 16,), jnp.float32)
    ones_c = jnp.ones((CHUNK, 8), jnp.float32)
    x2_pad = jnp.pad(x.reshape(n * 2), (0, (npad - n) * 2))

    deg_p, xtab = _deg_kernel(npad, e_pad)(dst, ones_c, zeros_s, zeros_1,
                                           x2_pad)

    disf, xnf = _prep_call(npad, deg_p.reshape(NC, G, 128),
                           xtab.reshape(G, 128))

    t1_p = _gs_kernel(npad, e_pad)(src, dst, xnf.reshape(npad, 8), zeros_s)

    gnf = _mid_call(npad, t1_p.reshape(NC, G, 128), xnf, disf,
                    W1big, b1big, W2big)

    t2_p = _gs_kernel(npad, e_pad)(src, dst, gnf.reshape(npad, 8), zeros_s)

    agg2f = _post_call(npad, t2_p.reshape(NC, G, 128), gnf, disf)
    return _final_call(n, agg2f.reshape(npad, 8), fcW, b2, fcb)


# speedup vs baseline: 1.3945x; 1.0263x over previous
import functools
import jax
import jax.numpy as jnp
from jax import lax
from jax.experimental import pallas as pl
from jax.experimental.pallas import tpu as pltpu
from jax.experimental.pallas import tpu_sc as plsc

NC = 2
NS = 16
NT = NC * NS
CHUNK = 4000

_MESH = dict(core_axis_name="c", subcore_axis_name="s", num_cores=NC,
             num_subcores=NS)


def _round_up(a, m):
    return (a + m - 1) // m * m


ZR = 392


def _gs_kernel(npad, e_pad):
    F = 8
    per_tile = e_pad // NT
    n_chunks = per_tile // CHUNK
    slice_sz = npad // NS

    scratch = (
        [pltpu.VMEM((CHUNK,), jnp.int32) for _ in range(2)] +
        [pltpu.VMEM((CHUNK,), jnp.int32) for _ in range(2)] +
        [pltpu.VMEM((CHUNK, F), jnp.float32) for _ in range(2)] +
        [pltpu.VMEM_SHARED((npad, F), jnp.float32),
         pltpu.SemaphoreType.DMA, pltpu.SemaphoreType.DMA,
         pltpu.SemaphoreType.DMA, pltpu.SemaphoreType.DMA]
    )

    @functools.partial(
        pl.kernel,
        out_type=jax.ShapeDtypeStruct((NC, npad, F), jnp.float32),
        mesh=plsc.VectorSubcoreMesh(**_MESH),
        scratch_types=scratch,
        compiler_params=pltpu.CompilerParams(use_tc_tiling_on_sc=False),
    )
    def k(src_hbm, dst_hbm, table_hbm, zeros_hbm, out_hbm, *refs):
        sidx = refs[0:2]
        didx = refs[2:4]
        rows = refs[4:6]
        acc_sh = refs[6]
        gsem = refs[7:9]
        isem = refs[9:11]
        c = lax.axis_index("c")
        s = lax.axis_index("s")
        sl = pl.ds(s * slice_sz, slice_sz)
        for r in range(slice_sz // ZR):
            pltpu.sync_copy(zeros_hbm,
                            acc_sh.at[pl.ds(s * slice_sz + r * ZR, ZR)])
        plsc.subcore_barrier()

        base = (c * NS + s) * per_tile

        def off(kk):
            return pl.ds(base + kk * CHUNK, CHUNK)

        def idx_start(kk, b):
            pltpu.async_copy(dst_hbm.at[off(kk)], didx[b], isem[b])
            pltpu.async_copy(src_hbm.at[off(kk)], sidx[b], isem[b])

        def idx_wait(kk, b):
            pltpu.make_async_copy(dst_hbm.at[off(kk)], didx[b], isem[b]).wait()
            pltpu.make_async_copy(src_hbm.at[off(kk)], sidx[b],
                                  isem[b]).wait()

        def gather_start(b):
            pltpu.async_copy(table_hbm.at[sidx[b]], rows[b], gsem[b])

        def gather_wait(b):
            pltpu.make_async_copy(table_hbm.at[sidx[b]], rows[b],
                                  gsem[b]).wait()

        idx_start(0, 0)
        idx_wait(0, 0)
        gather_start(0)
        if n_chunks > 1:
            idx_start(1, 1)
        for kk in range(n_chunks):
            b = kk & 1
            nb = 1 - b
            gather_wait(b)
            if kk + 1 < n_chunks:
                idx_wait(kk + 1, nb)
                gather_start(nb)
            pltpu.sync_copy(rows[b], acc_sh.at[didx[b]], add=True)
            if kk + 2 < n_chunks:
                idx_start(kk + 2, b)

        plsc.subcore_barrier()
        pltpu.sync_copy(acc_sh.at[sl], out_hbm.at[c].at[sl])

    return k


def _deg_kernel(npad, e_pad):
    F = 8
    per_tile = e_pad // NT
    n_chunks = per_tile // CHUNK
    slice_sz = npad // NS

    scratch = (
        [pltpu.VMEM((CHUNK,), jnp.int32) for _ in range(2)] +
        [pltpu.VMEM((CHUNK, F), jnp.float32),
         pltpu.VMEM_SHARED((npad, F), jnp.float32),
         pltpu.SemaphoreType.DMA, pltpu.SemaphoreType.DMA]
    )

    @functools.partial(
        pl.kernel,
        out_type=jax.ShapeDtypeStruct((NC, npad, F), jnp.float32),
        mesh=plsc.VectorSubcoreMesh(**_MESH),
        scratch_types=scratch,
        compiler_params=pltpu.CompilerParams(use_tc_tiling_on_sc=False),
    )
    def k(dst_hbm, ones_hbm, zeros_hbm, out_hbm, *refs):
        didx = refs[0:2]
        ones_v = refs[2]
        acc_sh = refs[3]
        isem = refs[4:6]
        c = lax.axis_index("c")
        s = lax.axis_index("s")
        sl = pl.ds(s * slice_sz, slice_sz)
        for r in range(slice_sz // ZR):
            pltpu.sync_copy(zeros_hbm,
                            acc_sh.at[pl.ds(s * slice_sz + r * ZR, ZR)])
        pltpu.sync_copy(ones_hbm, ones_v)
        plsc.subcore_barrier()
        base = (c * NS + s) * per_tile

        def off(kk):
            return pl.ds(base + kk * CHUNK, CHUNK)

        def idx_start(kk, b):
            pltpu.async_copy(dst_hbm.at[off(kk)], didx[b], isem[b])

        def idx_wait(kk, b):
            pltpu.make_async_copy(dst_hbm.at[off(kk)], didx[b], isem[b]).wait()

        idx_start(0, 0)
        if n_chunks > 1:
            idx_start(1, 1)
        for kk in range(n_chunks):
            b = kk & 1
            idx_wait(kk, b)
            pltpu.sync_copy(ones_v, acc_sh.at[didx[b]], add=True)
            if kk + 2 < n_chunks:
                idx_start(kk + 2, b)

        plsc.subcore_barrier()
        pltpu.sync_copy(acc_sh.at[sl], out_hbm.at[c].at[sl])

    return k


def _prep_call(npad, deg_p, x_flat):
    def body(degp, xf, disf, xnf):
        deg = degp[0] + degp[1] + 1.0
        d = lax.rsqrt(deg)
        disf[...] = d
        xnf[...] = xf[...] * d

    G = npad // 16
    R = 784
    g = G // R
    return pl.pallas_call(
        body,
        grid=(g,),
        in_specs=[
            pl.BlockSpec((NC, R, 128), lambda i: (0, i, 0)),
            pl.BlockSpec((R, 128), lambda i: (i, 0)),
        ],
        out_specs=(
            pl.BlockSpec((R, 128), lambda i: (i, 0)),
            pl.BlockSpec((R, 128), lambda i: (i, 0)),
        ),
        out_shape=(
            jax.ShapeDtypeStruct((G, 128), jnp.float32),
            jax.ShapeDtypeStruct((G, 128), jnp.float32),
        ),
    )(deg_p, x_flat)


def _mid_call(npad, t1_p, xnf, disf, W1big, b1big, W2big):
    G = npad // 16
    R = 784
    g = G // R

    def body(tp, xnb, disb, w1, bb1, w2, gnf):
        agg = (tp[0] + tp[1] + xnb[...]) * disb[...]
        h1 = jnp.dot(agg, w1[...], preferred_element_type=jnp.float32)
        h1 = jnp.maximum(h1 + bb1[...], 0.0)
        gnf[...] = jnp.dot(h1, w2[...],
                           preferred_element_type=jnp.float32) * disb[...]

    return pl.pallas_call(
        body,
        grid=(g,),
        in_specs=[
            pl.BlockSpec((NC, R, 128), lambda i: (0, i, 0)),
            pl.BlockSpec((R, 128), lambda i: (i, 0)),
            pl.BlockSpec((R, 128), lambda i: (i, 0)),
            pl.BlockSpec((128, 256), lambda i: (0, 0)),
            pl.BlockSpec((1, 256), lambda i: (0, 0)),
            pl.BlockSpec((256, 128), lambda i: (0, 0)),
        ],
        out_specs=pl.BlockSpec((R, 128), lambda i: (i, 0)),
        out_shape=jax.ShapeDtypeStruct((G, 128), jnp.float32),
    )(t1_p, xnf, disf, W1big, b1big.reshape(1, 256), W2big)


def _post_call(npad, t2_p, gnf, disf):
    G = npad // 16
    R = 784
    g = G // R

    def body(tp, gnb, disb, aggf):
        aggf[...] = (tp[0] + tp[1] + gnb[...]) * disb[...]

    return pl.pallas_call(
        body,
        grid=(g,),
        in_specs=[
            pl.BlockSpec((NC, R, 128), lambda i: (0, i, 0)),
            pl.BlockSpec((R, 128), lambda i: (i, 0)),
            pl.BlockSpec((R, 128), lambda i: (i, 0)),
        ],
        out_specs=pl.BlockSpec((R, 128), lambda i: (i, 0)),
        out_shape=jax.ShapeDtypeStruct((G, 128), jnp.float32),
    )(t2_p, gnf, disf)


def _final_call(n, agg2, fcW, b2, fcb):
    R = 4000
    g = n // R

    def body(ab, w, bb2, bfc, out):
        cvec = jnp.dot(bb2[...], w[...],
                       preferred_element_type=jnp.float32) + bfc[...]
        out[...] = jnp.dot(ab[...], w[...],
                           preferred_element_type=jnp.float32) + cvec

    return pl.pallas_call(
        body,
        grid=(g,),
        in_specs=[
            pl.BlockSpec((R, 8), lambda i: (i, 0)),
            pl.BlockSpec((8, 256), lambda i: (0, 0)),
            pl.BlockSpec((1, 8), lambda i: (0, 0)),
            pl.BlockSpec((1, 256), lambda i: (0, 0)),
        ],
        out_specs=pl.BlockSpec((R, 256), lambda i: (i, 0)),
        out_shape=jax.ShapeDtypeStruct((n, 256), jnp.float32),
    )(agg2, fcW, b2.reshape(1, 8), fcb.reshape(1, 256))


@jax.jit
def kernel(x, edge_index, W1, b1, W2, b2, fcW, fcb):
    n = x.shape[0]
    e = edge_index.shape[1]
    npad = _round_up(n + 1, 2048)
    e_pad = _round_up(e, NT * CHUNK)

    src = edge_index[0]
    dst = edge_index[1]
    if e_pad != e:
        pad = jnp.full((e_pad - e,), n, dtype=jnp.int32)
        src = jnp.concatenate([src, pad])
        dst = jnp.concatenate([dst, pad])

    G = npad // 16
    W1p = jnp.zeros((8, 16), jnp.float32).at[:2].set(W1)
    W1big = jnp.kron(jnp.eye(16, dtype=jnp.float32), W1p)
    W2big = jnp.kron(jnp.eye(16, dtype=jnp.float32), W2)
    b1big = jnp.tile(b1, 16)
    zeros_s = jnp.zeros((ZR, 8), jnp.float32)
    ones_c = jnp.ones((CHUNK, 8), jnp.float32)
    x_flat = jnp.zeros((npad, 8), jnp.float32).at[:n, :2].set(x)
    x_flat = x_flat.reshape(G, 128)

    deg_p = _deg_kernel(npad, e_pad)(dst, ones_c, zeros_s)

    disf, xnf = _prep_call(npad, deg_p.reshape(NC, G, 128), x_flat)

    t1_p = _gs_kernel(npad, e_pad)(src, dst, xnf.reshape(npad, 8), zeros_s)

    gnf = _mid_call(npad, t1_p.reshape(NC, G, 128), xnf, disf,
                    W1big, b1big, W2big)

    t2_p = _gs_kernel(npad, e_pad)(src, dst, gnf.reshape(npad, 8), zeros_s)

    agg2f = _post_call(npad, t2_p.reshape(NC, G, 128), gnf, disf)
    return _final_call(n, agg2f.reshape(npad, 8), fcW, b2, fcb)


# speedup vs baseline: 1.5929x; 1.1423x over previous
import functools
import jax
import jax.numpy as jnp
import numpy as np
from jax import lax
from jax.experimental import pallas as pl
from jax.experimental.pallas import tpu as pltpu
from jax.experimental.pallas import tpu_sc as plsc

NC = 2
NS = 16
NT = NC * NS
CHUNK = 4000

_MESH = dict(core_axis_name="c", subcore_axis_name="s", num_cores=NC,
             num_subcores=NS)


def _round_up(a, m):
    return (a + m - 1) // m * m


ZR = 392


def _gs_kernel(npad, e_pad):
    F = 8
    per_tile = e_pad // NT
    n_chunks = per_tile // CHUNK
    slice_sz = npad // NS

    scratch = (
        [pltpu.VMEM((CHUNK,), jnp.int32) for _ in range(2)] +
        [pltpu.VMEM((CHUNK,), jnp.int32) for _ in range(2)] +
        [pltpu.VMEM((CHUNK, F), jnp.float32) for _ in range(2)] +
        [pltpu.VMEM_SHARED((npad, F), jnp.float32),
         pltpu.SemaphoreType.DMA, pltpu.SemaphoreType.DMA,
         pltpu.SemaphoreType.DMA, pltpu.SemaphoreType.DMA]
    )

    @functools.partial(
        pl.kernel,
        out_type=jax.ShapeDtypeStruct((NC, npad, F), jnp.float32),
        mesh=plsc.VectorSubcoreMesh(**_MESH),
        scratch_types=scratch,
        compiler_params=pltpu.CompilerParams(use_tc_tiling_on_sc=False),
    )
    def k(src_hbm, dst_hbm, table_hbm, zeros_hbm, out_hbm, *refs):
        sidx = refs[0:2]
        didx = refs[2:4]
        rows = refs[4:6]
        acc_sh = refs[6]
        gsem = refs[7:9]
        isem = refs[9:11]
        c = lax.axis_index("c")
        s = lax.axis_index("s")
        sl = pl.ds(s * slice_sz, slice_sz)
        pltpu.sync_copy(zeros_hbm, acc_sh.at[sl])
        plsc.subcore_barrier()

        base = (c * NS + s) * per_tile

        def off(kk):
            return pl.ds(base + kk * CHUNK, CHUNK)

        def idx_start(kk, b):
            pltpu.async_copy(dst_hbm.at[off(kk)], didx[b], isem[b])
            pltpu.async_copy(src_hbm.at[off(kk)], sidx[b], isem[b])

        def idx_wait(kk, b):
            pltpu.make_async_copy(dst_hbm.at[off(kk)], didx[b], isem[b]).wait()
            pltpu.make_async_copy(src_hbm.at[off(kk)], sidx[b],
                                  isem[b]).wait()

        def gather_start(b):
            pltpu.async_copy(table_hbm.at[sidx[b]], rows[b], gsem[b])

        def gather_wait(b):
            pltpu.make_async_copy(table_hbm.at[sidx[b]], rows[b],
                                  gsem[b]).wait()

        idx_start(0, 0)
        idx_wait(0, 0)
        gather_start(0)
        if n_chunks > 1:
            idx_start(1, 1)
        for kk in range(n_chunks):
            b = kk & 1
            nb = 1 - b
            gather_wait(b)
            if kk + 1 < n_chunks:
                idx_wait(kk + 1, nb)
                gather_start(nb)
            pltpu.sync_copy(rows[b], acc_sh.at[didx[b]], add=True)
            if kk + 2 < n_chunks:
                idx_start(kk + 2, b)

        plsc.subcore_barrier()
        pltpu.sync_copy(acc_sh.at[sl], out_hbm.at[c].at[sl])

    return k


def _deg_kernel(npad, e_pad):
    F = 8
    per_tile = e_pad // NT
    n_chunks = per_tile // CHUNK
    slice_sz = npad // NS

    scratch = (
        [pltpu.VMEM((CHUNK,), jnp.int32) for _ in range(2)] +
        [pltpu.VMEM((CHUNK, F), jnp.float32),
         pltpu.VMEM_SHARED((npad, F), jnp.float32),
         pltpu.SemaphoreType.DMA, pltpu.SemaphoreType.DMA]
    )

    @functools.partial(
        pl.kernel,
        out_type=jax.ShapeDtypeStruct((NC, npad, F), jnp.float32),
        mesh=plsc.VectorSubcoreMesh(**_MESH),
        scratch_types=scratch,
        compiler_params=pltpu.CompilerParams(use_tc_tiling_on_sc=False),
    )
    def k(dst_hbm, ones_hbm, zeros_hbm, out_hbm, *refs):
        didx = refs[0:2]
        ones_v = refs[2]
        acc_sh = refs[3]
        isem = refs[4:6]
        c = lax.axis_index("c")
        s = lax.axis_index("s")
        sl = pl.ds(s * slice_sz, slice_sz)
        pltpu.sync_copy(zeros_hbm, acc_sh.at[sl])
        pltpu.sync_copy(ones_hbm, ones_v)
        plsc.subcore_barrier()
        base = (c * NS + s) * per_tile

        def off(kk):
            return pl.ds(base + kk * CHUNK, CHUNK)

        def idx_start(kk, b):
            pltpu.async_copy(dst_hbm.at[off(kk)], didx[b], isem[b])

        def idx_wait(kk, b):
            pltpu.make_async_copy(dst_hbm.at[off(kk)], didx[b], isem[b]).wait()

        idx_start(0, 0)
        if n_chunks > 1:
            idx_start(1, 1)
        for kk in range(n_chunks):
            b = kk & 1
            idx_wait(kk, b)
            pltpu.sync_copy(ones_v, acc_sh.at[didx[b]], add=True)
            if kk + 2 < n_chunks:
                idx_start(kk + 2, b)

        plsc.subcore_barrier()
        pltpu.sync_copy(acc_sh.at[sl], out_hbm.at[c].at[sl])

    return k


_SPREAD = np.zeros((32, 128), np.float32)
for _j in range(16):
    for _k in range(2):
        _SPREAD[2 * _j + _k, 8 * _j + _k] = 1.0


def _prep_call(npad, deg_p, x32):
    def body(degp, xr, spread, disf, xnf):
        deg = degp[0] + degp[1] + 1.0
        d = lax.rsqrt(deg)
        disf[...] = d
        xf = jnp.dot(xr[...], spread[...], preferred_element_type=jnp.float32)
        xnf[...] = xf * d

    G = npad // 16
    R = 784
    g = G // R
    return pl.pallas_call(
        body,
        grid=(g,),
        in_specs=[
            pl.BlockSpec((NC, R, 128), lambda i: (0, i, 0)),
            pl.BlockSpec((R, 32), lambda i: (i, 0)),
            pl.BlockSpec((32, 128), lambda i: (0, 0)),
        ],
        out_specs=(
            pl.BlockSpec((R, 128), lambda i: (i, 0)),
            pl.BlockSpec((R, 128), lambda i: (i, 0)),
        ),
        out_shape=(
            jax.ShapeDtypeStruct((G, 128), jnp.float32),
            jax.ShapeDtypeStruct((G, 128), jnp.float32),
        ),
    )(deg_p, x32, _SPREAD)


def _mid_call(npad, t1_p, xnf, disf, W1big, b1big, W2big):
    G = npad // 16
    R = 784
    g = G // R

    def body(tp, xnb, disb, w1, bb1, w2, gnf):
        agg = (tp[0] + tp[1] + xnb[...]) * disb[...]
        h1 = jnp.dot(agg, w1[...], preferred_element_type=jnp.float32)
        h1 = jnp.maximum(h1 + bb1[...], 0.0)
        gnf[...] = jnp.dot(h1, w2[...],
                           preferred_element_type=jnp.float32) * disb[...]

    return pl.pallas_call(
        body,
        grid=(g,),
        in_specs=[
            pl.BlockSpec((NC, R, 128), lambda i: (0, i, 0)),
            pl.BlockSpec((R, 128), lambda i: (i, 0)),
            pl.BlockSpec((R, 128), lambda i: (i, 0)),
            pl.BlockSpec((128, 256), lambda i: (0, 0)),
            pl.BlockSpec((1, 256), lambda i: (0, 0)),
            pl.BlockSpec((256, 128), lambda i: (0, 0)),
        ],
        out_specs=pl.BlockSpec((R, 128), lambda i: (i, 0)),
        out_shape=jax.ShapeDtypeStruct((G, 128), jnp.float32),
    )(t1_p, xnf, disf, W1big, b1big.reshape(1, 256), W2big)


def _post_call(npad, t2_p, gnf, disf):
    G = npad // 16
    R = 784
    g = G // R

    def body(tp, gnb, disb, aggf):
        aggf[...] = (tp[0] + tp[1] + gnb[...]) * disb[...]

    return pl.pallas_call(
        body,
        grid=(g,),
        in_specs=[
            pl.BlockSpec((NC, R, 128), lambda i: (0, i, 0)),
            pl.BlockSpec((R, 128), lambda i: (i, 0)),
            pl.BlockSpec((R, 128), lambda i: (i, 0)),
        ],
        out_specs=pl.BlockSpec((R, 128), lambda i: (i, 0)),
        out_shape=jax.ShapeDtypeStruct((G, 128), jnp.float32),
    )(t2_p, gnf, disf)


def _final_call(n, agg2, fcW, b2, fcb):
    R = 4000
    g = n // R

    def body(ab, w, bb2, bfc, out):
        cvec = jnp.dot(bb2[...], w[...],
                       preferred_element_type=jnp.float32) + bfc[...]
        out[...] = jnp.dot(ab[...], w[...],
                           preferred_element_type=jnp.float32) + cvec

    return pl.pallas_call(
        body,
        grid=(g,),
        in_specs=[
            pl.BlockSpec((R, 8), lambda i: (i, 0)),
            pl.BlockSpec((8, 256), lambda i: (0, 0)),
            pl.BlockSpec((1, 8), lambda i: (0, 0)),
            pl.BlockSpec((1, 256), lambda i: (0, 0)),
        ],
        out_specs=pl.BlockSpec((R, 256), lambda i: (i, 0)),
        out_shape=jax.ShapeDtypeStruct((n, 256), jnp.float32),
    )(agg2, fcW, b2.reshape(1, 8), fcb.reshape(1, 256))


@jax.jit
def kernel(x, edge_index, W1, b1, W2, b2, fcW, fcb):
    n = x.shape[0]
    e = edge_index.shape[1]
    npad = _round_up(n + 1, 2048)
    e_pad = _round_up(e, NT * CHUNK)

    src = edge_index[0]
    dst = edge_index[1]
    if e_pad != e:
        pad = jnp.full((e_pad - e,), n, dtype=jnp.int32)
        src = jnp.concatenate([src, pad])
        dst = jnp.concatenate([dst, pad])

    G = npad // 16
    W1p = jnp.zeros((8, 16), jnp.float32).at[:2].set(W1)
    W1big = jnp.kron(jnp.eye(16, dtype=jnp.float32), W1p)
    W2big = jnp.kron(jnp.eye(16, dtype=jnp.float32), W2)
    b1big = jnp.tile(b1, 16)
    zeros_s = np.zeros((npad // NS, 8), np.float32)
    ones_c = np.ones((CHUNK, 8), np.float32)
    x32 = jnp.pad(x.reshape(n * 2), (0, (npad - n) * 2)).reshape(G, 32)

    deg_p = _deg_kernel(npad, e_pad)(dst, ones_c, zeros_s)

    disf, xnf = _prep_call(npad, deg_p.reshape(NC, G, 128), x32)

    t1_p = _gs_kernel(npad, e_pad)(src, dst, xnf.reshape(npad, 8), zeros_s)

    gnf = _mid_call(npad, t1_p.reshape(NC, G, 128), xnf, disf,
                    W1big, b1big, W2big)

    t2_p = _gs_kernel(npad, e_pad)(src, dst, gnf.reshape(npad, 8), zeros_s)

    agg2f = _post_call(npad, t2_p.reshape(NC, G, 128), gnf, disf)
    return _final_call(n, agg2f.reshape(npad, 8), fcW, b2, fcb)


# speedup vs baseline: 1.5980x; 1.0032x over previous
import functools
import jax
import jax.numpy as jnp
import numpy as np
from jax import lax
from jax.experimental import pallas as pl
from jax.experimental.pallas import tpu as pltpu
from jax.experimental.pallas import tpu_sc as plsc

NC = 2
NS = 16
NT = NC * NS
CHUNK = 4000

_MESH = dict(core_axis_name="c", subcore_axis_name="s", num_cores=NC,
             num_subcores=NS)


def _round_up(a, m):
    return (a + m - 1) // m * m


ZR = 392


def _gs_kernel(npad, e_pad, spmem_table=False, chunk=CHUNK):
    F = 8
    per_tile = e_pad // NT
    n_chunks = per_tile // chunk
    slice_sz = npad // NS

    scratch = (
        [pltpu.VMEM((chunk,), jnp.int32) for _ in range(2)] +
        [pltpu.VMEM((chunk,), jnp.int32) for _ in range(2)] +
        [pltpu.VMEM((chunk, F), jnp.float32) for _ in range(2)] +
        [pltpu.VMEM_SHARED((npad, F), jnp.float32),
         pltpu.SemaphoreType.DMA, pltpu.SemaphoreType.DMA,
         pltpu.SemaphoreType.DMA, pltpu.SemaphoreType.DMA]
        + ([pltpu.VMEM_SHARED((npad, F), jnp.float32)]
           if spmem_table else [])
    )

    @functools.partial(
        pl.kernel,
        out_type=jax.ShapeDtypeStruct((NC, npad, F), jnp.float32),
        mesh=plsc.VectorSubcoreMesh(**_MESH),
        scratch_types=scratch,
        compiler_params=pltpu.CompilerParams(use_tc_tiling_on_sc=False),
    )
    def k(src_hbm, dst_hbm, table_hbm, zeros_hbm, out_hbm, *refs):
        sidx = refs[0:2]
        didx = refs[2:4]
        rows = refs[4:6]
        acc_sh = refs[6]
        gsem = refs[7:9]
        isem = refs[9:11]
        c = lax.axis_index("c")
        s = lax.axis_index("s")
        sl = pl.ds(s * slice_sz, slice_sz)
        pltpu.sync_copy(zeros_hbm, acc_sh.at[sl])
        if spmem_table:
            tbl = refs[11]
            pltpu.sync_copy(table_hbm.at[sl], tbl.at[sl])
        else:
            tbl = table_hbm
        plsc.subcore_barrier()

        base = (c * NS + s) * per_tile

        def off(kk):
            return pl.ds(base + kk * chunk, chunk)

        def idx_start(kk, b):
            pltpu.async_copy(dst_hbm.at[off(kk)], didx[b], isem[b])
            pltpu.async_copy(src_hbm.at[off(kk)], sidx[b], isem[b])

        def idx_wait(kk, b):
            pltpu.make_async_copy(dst_hbm.at[off(kk)], didx[b], isem[b]).wait()
            pltpu.make_async_copy(src_hbm.at[off(kk)], sidx[b],
                                  isem[b]).wait()

        def gather_start(b):
            pltpu.async_copy(tbl.at[sidx[b]], rows[b], gsem[b])

        def gather_wait(b):
            pltpu.make_async_copy(tbl.at[sidx[b]], rows[b],
                                  gsem[b]).wait()

        idx_start(0, 0)
        idx_wait(0, 0)
        gather_start(0)
        if n_chunks > 1:
            idx_start(1, 1)
        for kk in range(n_chunks):
            b = kk & 1
            nb = 1 - b
            gather_wait(b)
            if kk + 1 < n_chunks:
                idx_wait(kk + 1, nb)
                gather_start(nb)
            pltpu.sync_copy(rows[b], acc_sh.at[didx[b]], add=True)
            if kk + 2 < n_chunks:
                idx_start(kk + 2, b)

        plsc.subcore_barrier()
        pltpu.sync_copy(acc_sh.at[sl], out_hbm.at[c].at[sl])

    return k


def _deg_kernel(npad, e_pad):
    F = 8
    per_tile = e_pad // NT
    n_chunks = per_tile // CHUNK
    slice_sz = npad // NS

    scratch = (
        [pltpu.VMEM((CHUNK,), jnp.int32) for _ in range(2)] +
        [pltpu.VMEM((CHUNK, F), jnp.float32),
         pltpu.VMEM_SHARED((npad, F), jnp.float32),
         pltpu.SemaphoreType.DMA, pltpu.SemaphoreType.DMA]
    )

    @functools.partial(
        pl.kernel,
        out_type=jax.ShapeDtypeStruct((NC, npad, F), jnp.float32),
        mesh=plsc.VectorSubcoreMesh(**_MESH),
        scratch_types=scratch,
        compiler_params=pltpu.CompilerParams(use_tc_tiling_on_sc=False),
    )
    def k(dst_hbm, ones_hbm, zeros_hbm, out_hbm, *refs):
        didx = refs[0:2]
        ones_v = refs[2]
        acc_sh = refs[3]
        isem = refs[4:6]
        c = lax.axis_index("c")
        s = lax.axis_index("s")
        sl = pl.ds(s * slice_sz, slice_sz)
        pltpu.sync_copy(zeros_hbm, acc_sh.at[sl])
        pltpu.sync_copy(ones_hbm, ones_v)
        plsc.subcore_barrier()
        base = (c * NS + s) * per_tile

        def off(kk):
            return pl.ds(base + kk * CHUNK, CHUNK)

        def idx_start(kk, b):
            pltpu.async_copy(dst_hbm.at[off(kk)], didx[b], isem[b])

        def idx_wait(kk, b):
            pltpu.make_async_copy(dst_hbm.at[off(kk)], didx[b], isem[b]).wait()

        idx_start(0, 0)
        if n_chunks > 1:
            idx_start(1, 1)
        for kk in range(n_chunks):
            b = kk & 1
            idx_wait(kk, b)
            pltpu.sync_copy(ones_v, acc_sh.at[didx[b]], add=True)
            if kk + 2 < n_chunks:
                idx_start(kk + 2, b)

        plsc.subcore_barrier()
        pltpu.sync_copy(acc_sh.at[sl], out_hbm.at[c].at[sl])

    return k


_SPREAD = np.zeros((32, 128), np.float32)
for _j in range(16):
    for _k in range(2):
        _SPREAD[2 * _j + _k, 8 * _j + _k] = 1.0


def _prep_call(npad, deg_p, x32):
    def body(degp, xr, spread, disf, xnf):
        deg = degp[0] + degp[1] + 1.0
        d = lax.rsqrt(deg)
        disf[...] = d
        xf = jnp.dot(xr[...], spread[...], preferred_element_type=jnp.float32)
        xnf[...] = xf * d

    G = npad // 16
    R = 784
    g = G // R
    return pl.pallas_call(
        body,
        grid=(g,),
        in_specs=[
            pl.BlockSpec((NC, R, 128), lambda i: (0, i, 0)),
            pl.BlockSpec((R, 32), lambda i: (i, 0)),
            pl.BlockSpec((32, 128), lambda i: (0, 0)),
        ],
        out_specs=(
            pl.BlockSpec((R, 128), lambda i: (i, 0)),
            pl.BlockSpec((R, 128), lambda i: (i, 0)),
        ),
        out_shape=(
            jax.ShapeDtypeStruct((G, 128), jnp.float32),
            jax.ShapeDtypeStruct((G, 128), jnp.float32),
        ),
    )(deg_p, x32, _SPREAD)


def _mid_call(npad, t1_p, xnf, disf, W1big, b1big, W2big):
    G = npad // 16
    R = 784
    g = G // R

    def body(tp, xnb, disb, w1, bb1, w2, gnf):
        agg = (tp[0] + tp[1] + xnb[...]) * disb[...]
        h1 = jnp.dot(agg, w1[...], preferred_element_type=jnp.float32)
        h1 = jnp.maximum(h1 + bb1[...], 0.0)
        gnf[...] = jnp.dot(h1, w2[...],
                           preferred_element_type=jnp.float32) * disb[...]

    return pl.pallas_call(
        body,
        grid=(g,),
        in_specs=[
            pl.BlockSpec((NC, R, 128), lambda i: (0, i, 0)),
            pl.BlockSpec((R, 128), lambda i: (i, 0)),
            pl.BlockSpec((R, 128), lambda i: (i, 0)),
            pl.BlockSpec((128, 256), lambda i: (0, 0)),
            pl.BlockSpec((1, 256), lambda i: (0, 0)),
            pl.BlockSpec((256, 128), lambda i: (0, 0)),
        ],
        out_specs=pl.BlockSpec((R, 128), lambda i: (i, 0)),
        out_shape=jax.ShapeDtypeStruct((G, 128), jnp.float32),
    )(t1_p, xnf, disf, W1big, b1big.reshape(1, 256), W2big)


def _post_call(npad, t2_p, gnf, disf):
    G = npad // 16
    R = 784
    g = G // R

    def body(tp, gnb, disb, aggf):
        aggf[...] = (tp[0] + tp[1] + gnb[...]) * disb[...]

    return pl.pallas_call(
        body,
        grid=(g,),
        in_specs=[
            pl.BlockSpec((NC, R, 128), lambda i: (0, i, 0)),
            pl.BlockSpec((R, 128), lambda i: (i, 0)),
            pl.BlockSpec((R, 128), lambda i: (i, 0)),
        ],
        out_specs=pl.BlockSpec((R, 128), lambda i: (i, 0)),
        out_shape=jax.ShapeDtypeStruct((G, 128), jnp.float32),
    )(t2_p, gnf, disf)


def _final_call(n, agg2, fcW, b2, fcb):
    R = 4000
    g = n // R

    def body(ab, w, bb2, bfc, out):
        cvec = jnp.dot(bb2[...], w[...],
                       preferred_element_type=jnp.float32) + bfc[...]
        out[...] = jnp.dot(ab[...], w[...],
                           preferred_element_type=jnp.float32) + cvec

    return pl.pallas_call(
        body,
        grid=(g,),
        in_specs=[
            pl.BlockSpec((R, 8), lambda i: (i, 0)),
            pl.BlockSpec((8, 256), lambda i: (0, 0)),
            pl.BlockSpec((1, 8), lambda i: (0, 0)),
            pl.BlockSpec((1, 256), lambda i: (0, 0)),
        ],
        out_specs=pl.BlockSpec((R, 256), lambda i: (i, 0)),
        out_shape=jax.ShapeDtypeStruct((n, 256), jnp.float32),
    )(agg2, fcW, b2.reshape(1, 8), fcb.reshape(1, 256))


@jax.jit
def kernel(x, edge_index, W1, b1, W2, b2, fcW, fcb):
    n = x.shape[0]
    e = edge_index.shape[1]
    npad = _round_up(n + 1, 2048)
    e_pad = _round_up(e, NT * CHUNK)

    src = edge_index[0]
    dst = edge_index[1]
    if e_pad != e:
        pad = jnp.full((e_pad - e,), n, dtype=jnp.int32)
        src = jnp.concatenate([src, pad])
        dst = jnp.concatenate([dst, pad])

    G = npad // 16
    W1p = jnp.zeros((8, 16), jnp.float32).at[:2].set(W1)
    W1big = jnp.kron(jnp.eye(16, dtype=jnp.float32), W1p)
    W2big = jnp.kron(jnp.eye(16, dtype=jnp.float32), W2)
    b1big = jnp.tile(b1, 16)
    zeros_s = np.zeros((npad // NS, 8), np.float32)
    ones_c = np.ones((CHUNK, 8), np.float32)
    x32 = jnp.pad(x.reshape(n * 2), (0, (npad - n) * 2)).reshape(G, 32)

    deg_p = _deg_kernel(npad, e_pad)(dst, ones_c, zeros_s)

    disf, xnf = _prep_call(npad, deg_p.reshape(NC, G, 128), x32)

    t1_p = _gs_kernel(npad, e_pad, spmem_table=True, chunk=1000)(
        src, dst, xnf.reshape(npad, 8), zeros_s)

    gnf = _mid_call(npad, t1_p.reshape(NC, G, 128), xnf, disf,
                    W1big, b1big, W2big)

    t2_p = _gs_kernel(npad, e_pad)(src, dst, gnf.reshape(npad, 8), zeros_s)

    agg2f = _post_call(npad, t2_p.reshape(NC, G, 128), gnf, disf)
    return _final_call(n, agg2f.reshape(npad, 8), fcW, b2, fcb)


# speedup vs baseline: 1.6116x; 1.0085x over previous
import functools
import jax
import jax.numpy as jnp
import numpy as np
from jax import lax
from jax.experimental import pallas as pl
from jax.experimental.pallas import tpu as pltpu
from jax.experimental.pallas import tpu_sc as plsc

NC = 2
NS = 16
NT = NC * NS
CHUNK = 4000

_MESH = dict(core_axis_name="c", subcore_axis_name="s", num_cores=NC,
             num_subcores=NS)


def _round_up(a, m):
    return (a + m - 1) // m * m


ZR = 392


def _gs_kernel(npad, e_pad, spmem_table=False, chunk=CHUNK):
    F = 8
    per_tile = e_pad // NT
    n_chunks = per_tile // chunk
    slice_sz = npad // NS

    scratch = (
        [pltpu.VMEM((chunk,), jnp.int32) for _ in range(2)] +
        [pltpu.VMEM((chunk,), jnp.int32) for _ in range(2)] +
        [pltpu.VMEM((chunk, F), jnp.float32) for _ in range(2)] +
        [pltpu.VMEM_SHARED((npad, F), jnp.float32),
         pltpu.SemaphoreType.DMA, pltpu.SemaphoreType.DMA,
         pltpu.SemaphoreType.DMA, pltpu.SemaphoreType.DMA,
         pltpu.SemaphoreType.DMA, pltpu.SemaphoreType.DMA]
        + ([pltpu.VMEM_SHARED((npad, F), jnp.float32)]
           if spmem_table else [])
    )

    @functools.partial(
        pl.kernel,
        out_type=jax.ShapeDtypeStruct((NC, npad, F), jnp.float32),
        mesh=plsc.VectorSubcoreMesh(**_MESH),
        scratch_types=scratch,
        compiler_params=pltpu.CompilerParams(use_tc_tiling_on_sc=False),
    )
    def k(src_hbm, dst_hbm, table_hbm, zeros_hbm, out_hbm, *refs):
        sidx = refs[0:2]
        didx = refs[2:4]
        rows = refs[4:6]
        acc_sh = refs[6]
        gsem = refs[7:9]
        gsem2 = refs[9:11]
        isem = refs[11:13]
        c = lax.axis_index("c")
        s = lax.axis_index("s")
        sl = pl.ds(s * slice_sz, slice_sz)
        pltpu.sync_copy(zeros_hbm, acc_sh.at[sl])
        if spmem_table:
            tbl = refs[13]
            pltpu.sync_copy(table_hbm.at[sl], tbl.at[sl])
        else:
            tbl = table_hbm
        plsc.subcore_barrier()

        base = (c * NS + s) * per_tile
        h = chunk // 2

        def off(kk):
            return pl.ds(base + kk * chunk, chunk)

        def idx_start(kk, b):
            pltpu.async_copy(dst_hbm.at[off(kk)], didx[b], isem[b])
            pltpu.async_copy(src_hbm.at[off(kk)], sidx[b], isem[b])

        def idx_wait(kk, b):
            pltpu.make_async_copy(dst_hbm.at[off(kk)], didx[b], isem[b]).wait()
            pltpu.make_async_copy(src_hbm.at[off(kk)], sidx[b],
                                  isem[b]).wait()

        def gather_start(b):
            pltpu.async_copy(tbl.at[sidx[b].at[pl.ds(0, h)]],
                             rows[b].at[pl.ds(0, h)], gsem[b])
            pltpu.async_copy(tbl.at[sidx[b].at[pl.ds(h, h)]],
                             rows[b].at[pl.ds(h, h)], gsem2[b])

        def gather_wait(b):
            pltpu.make_async_copy(tbl.at[sidx[b].at[pl.ds(0, h)]],
                                  rows[b].at[pl.ds(0, h)], gsem[b]).wait()
            pltpu.make_async_copy(tbl.at[sidx[b].at[pl.ds(h, h)]],
                                  rows[b].at[pl.ds(h, h)], gsem2[b]).wait()

        idx_start(0, 0)
        idx_wait(0, 0)
        gather_start(0)
        if n_chunks > 1:
            idx_start(1, 1)
        for kk in range(n_chunks):
            b = kk & 1
            nb = 1 - b
            gather_wait(b)
            if kk + 1 < n_chunks:
                idx_wait(kk + 1, nb)
                gather_start(nb)
            pltpu.sync_copy(rows[b], acc_sh.at[didx[b]], add=True)
            if kk + 2 < n_chunks:
                idx_start(kk + 2, b)

        plsc.subcore_barrier()
        pltpu.sync_copy(acc_sh.at[sl], out_hbm.at[c].at[sl])

    return k


def _deg_kernel(npad, e_pad):
    F = 8
    per_tile = e_pad // NT
    n_chunks = per_tile // CHUNK
    slice_sz = npad // NS

    scratch = (
        [pltpu.VMEM((CHUNK,), jnp.int32) for _ in range(2)] +
        [pltpu.VMEM((CHUNK, F), jnp.float32),
         pltpu.VMEM_SHARED((npad, F), jnp.float32),
         pltpu.SemaphoreType.DMA, pltpu.SemaphoreType.DMA]
    )

    @functools.partial(
        pl.kernel,
        out_type=jax.ShapeDtypeStruct((NC, npad, F), jnp.float32),
        mesh=plsc.VectorSubcoreMesh(**_MESH),
        scratch_types=scratch,
        compiler_params=pltpu.CompilerParams(use_tc_tiling_on_sc=False),
    )
    def k(dst_hbm, ones_hbm, zeros_hbm, out_hbm, *refs):
        didx = refs[0:2]
        ones_v = refs[2]
        acc_sh = refs[3]
        isem = refs[4:6]
        c = lax.axis_index("c")
        s = lax.axis_index("s")
        sl = pl.ds(s * slice_sz, slice_sz)
        pltpu.sync_copy(zeros_hbm, acc_sh.at[sl])
        pltpu.sync_copy(ones_hbm, ones_v)
        plsc.subcore_barrier()
        base = (c * NS + s) * per_tile

        def off(kk):
            return pl.ds(base + kk * CHUNK, CHUNK)

        def idx_start(kk, b):
            pltpu.async_copy(dst_hbm.at[off(kk)], didx[b], isem[b])

        def idx_wait(kk, b):
            pltpu.make_async_copy(dst_hbm.at[off(kk)], didx[b], isem[b]).wait()

        idx_start(0, 0)
        if n_chunks > 1:
            idx_start(1, 1)
        for kk in range(n_chunks):
            b = kk & 1
            idx_wait(kk, b)
            pltpu.sync_copy(ones_v, acc_sh.at[didx[b]], add=True)
            if kk + 2 < n_chunks:
                idx_start(kk + 2, b)

        plsc.subcore_barrier()
        pltpu.sync_copy(acc_sh.at[sl], out_hbm.at[c].at[sl])

    return k


_SPREAD = np.zeros((32, 128), np.float32)
for _j in range(16):
    for _k in range(2):
        _SPREAD[2 * _j + _k, 8 * _j + _k] = 1.0


def _prep_call(npad, deg_p, x32):
    def body(degp, xr, spread, disf, xnf):
        deg = degp[0] + degp[1] + 1.0
        d = lax.rsqrt(deg)
        disf[...] = d
        xf = jnp.dot(xr[...], spread[...], preferred_element_type=jnp.float32)
        xnf[...] = xf * d

    G = npad // 16
    R = 784
    g = G // R
    return pl.pallas_call(
        body,
        grid=(g,),
        in_specs=[
            pl.BlockSpec((NC, R, 128), lambda i: (0, i, 0)),
            pl.BlockSpec((R, 32), lambda i: (i, 0)),
            pl.BlockSpec((32, 128), lambda i: (0, 0)),
        ],
        out_specs=(
            pl.BlockSpec((R, 128), lambda i: (i, 0)),
            pl.BlockSpec((R, 128), lambda i: (i, 0)),
        ),
        out_shape=(
            jax.ShapeDtypeStruct((G, 128), jnp.float32),
            jax.ShapeDtypeStruct((G, 128), jnp.float32),
        ),
    )(deg_p, x32, _SPREAD)


def _mid_call(npad, t1_p, xnf, disf, W1big, b1big, W2big):
    G = npad // 16
    R = 784
    g = G // R

    def body(tp, xnb, disb, w1, bb1, w2, gnf):
        agg = (tp[0] + tp[1] + xnb[...]) * disb[...]
        h1 = jnp.dot(agg, w1[...], preferred_element_type=jnp.float32)
        h1 = jnp.maximum(h1 + bb1[...], 0.0)
        gnf[...] = jnp.dot(h1, w2[...],
                           preferred_element_type=jnp.float32) * disb[...]

    return pl.pallas_call(
        body,
        grid=(g,),
        in_specs=[
            pl.BlockSpec((NC, R, 128), lambda i: (0, i, 0)),
            pl.BlockSpec((R, 128), lambda i: (i, 0)),
            pl.BlockSpec((R, 128), lambda i: (i, 0)),
            pl.BlockSpec((128, 256), lambda i: (0, 0)),
            pl.BlockSpec((1, 256), lambda i: (0, 0)),
            pl.BlockSpec((256, 128), lambda i: (0, 0)),
        ],
        out_specs=pl.BlockSpec((R, 128), lambda i: (i, 0)),
        out_shape=jax.ShapeDtypeStruct((G, 128), jnp.float32),
    )(t1_p, xnf, disf, W1big, b1big.reshape(1, 256), W2big)


def _post_call(npad, t2_p, gnf, disf):
    G = npad // 16
    R = 784
    g = G // R

    def body(tp, gnb, disb, aggf):
        aggf[...] = (tp[0] + tp[1] + gnb[...]) * disb[...]

    return pl.pallas_call(
        body,
        grid=(g,),
        in_specs=[
            pl.BlockSpec((NC, R, 128), lambda i: (0, i, 0)),
            pl.BlockSpec((R, 128), lambda i: (i, 0)),
            pl.BlockSpec((R, 128), lambda i: (i, 0)),
        ],
        out_specs=pl.BlockSpec((R, 128), lambda i: (i, 0)),
        out_shape=jax.ShapeDtypeStruct((G, 128), jnp.float32),
    )(t2_p, gnf, disf)


def _final_call(n, agg2, fcW, b2, fcb):
    R = 4000
    g = n // R

    def body(ab, w, bb2, bfc, out):
        cvec = jnp.dot(bb2[...], w[...],
                       preferred_element_type=jnp.float32) + bfc[...]
        out[...] = jnp.dot(ab[...], w[...],
                           preferred_element_type=jnp.float32) + cvec

    return pl.pallas_call(
        body,
        grid=(g,),
        in_specs=[
            pl.BlockSpec((R, 8), lambda i: (i, 0)),
            pl.BlockSpec((8, 256), lambda i: (0, 0)),
            pl.BlockSpec((1, 8), lambda i: (0, 0)),
            pl.BlockSpec((1, 256), lambda i: (0, 0)),
        ],
        out_specs=pl.BlockSpec((R, 256), lambda i: (i, 0)),
        out_shape=jax.ShapeDtypeStruct((n, 256), jnp.float32),
    )(agg2, fcW, b2.reshape(1, 8), fcb.reshape(1, 256))


@jax.jit
def kernel(x, edge_index, W1, b1, W2, b2, fcW, fcb):
    n = x.shape[0]
    e = edge_index.shape[1]
    npad = _round_up(n + 1, 2048)
    e_pad = _round_up(e, NT * CHUNK)

    src = edge_index[0]
    dst = edge_index[1]
    if e_pad != e:
        pad = jnp.full((e_pad - e,), n, dtype=jnp.int32)
        src = jnp.concatenate([src, pad])
        dst = jnp.concatenate([dst, pad])

    G = npad // 16
    W1p = jnp.zeros((8, 16), jnp.float32).at[:2].set(W1)
    W1big = jnp.kron(jnp.eye(16, dtype=jnp.float32), W1p)
    W2big = jnp.kron(jnp.eye(16, dtype=jnp.float32), W2)
    b1big = jnp.tile(b1, 16)
    zeros_s = np.zeros((npad // NS, 8), np.float32)
    ones_c = np.ones((CHUNK, 8), np.float32)
    x32 = jnp.pad(x.reshape(n * 2), (0, (npad - n) * 2)).reshape(G, 32)

    deg_p = _deg_kernel(npad, e_pad)(dst, ones_c, zeros_s)

    disf, xnf = _prep_call(npad, deg_p.reshape(NC, G, 128), x32)

    t1_p = _gs_kernel(npad, e_pad)(src, dst, xnf.reshape(npad, 8), zeros_s)

    gnf = _mid_call(npad, t1_p.reshape(NC, G, 128), xnf, disf,
                    W1big, b1big, W2big)

    t2_p = _gs_kernel(npad, e_pad)(src, dst, gnf.reshape(npad, 8), zeros_s)

    agg2f = _post_call(npad, t2_p.reshape(NC, G, 128), gnf, disf)
    return _final_call(n, agg2f.reshape(npad, 8), fcW, b2, fcb)


# speedup vs baseline: 1.6134x; 1.0012x over previous
import functools
import jax
import jax.numpy as jnp
import numpy as np
from jax import lax
from jax.experimental import pallas as pl
from jax.experimental.pallas import tpu as pltpu
from jax.experimental.pallas import tpu_sc as plsc

NC = 2
NS = 16
NT = NC * NS
CHUNK = 4000

_MESH = dict(core_axis_name="c", subcore_axis_name="s", num_cores=NC,
             num_subcores=NS)


def _round_up(a, m):
    return (a + m - 1) // m * m


def _gs_kernel(npad, e_pad, spmem_table=False, chunk=CHUNK):
    F = 8
    per_tile = e_pad // NT
    n_chunks = per_tile // chunk
    slice_sz = npad // NS

    scratch = (
        [pltpu.VMEM((chunk,), jnp.int32) for _ in range(2)] +
        [pltpu.VMEM((chunk,), jnp.int32) for _ in range(2)] +
        [pltpu.VMEM((chunk, F), jnp.float32) for _ in range(2)] +
        [pltpu.VMEM_SHARED((npad, F), jnp.float32),
         pltpu.SemaphoreType.DMA, pltpu.SemaphoreType.DMA,
         pltpu.SemaphoreType.DMA, pltpu.SemaphoreType.DMA,
         pltpu.SemaphoreType.DMA, pltpu.SemaphoreType.DMA]
        + ([pltpu.VMEM_SHARED((npad, F), jnp.float32)]
           if spmem_table else [])
    )

    @functools.partial(
        pl.kernel,
        out_type=jax.ShapeDtypeStruct((NC, npad, F), jnp.float32),
        mesh=plsc.VectorSubcoreMesh(**_MESH),
        scratch_types=scratch,
        compiler_params=pltpu.CompilerParams(use_tc_tiling_on_sc=False),
    )
    def k(src_hbm, dst_hbm, table_hbm, zeros_hbm, out_hbm, *refs):
        sidx = refs[0:2]
        didx = refs[2:4]
        rows = refs[4:6]
        acc_sh = refs[6]
        gsem = refs[7:9]
        gsem2 = refs[9:11]
        isem = refs[11:13]
        c = lax.axis_index("c")
        s = lax.axis_index("s")
        sl = pl.ds(s * slice_sz, slice_sz)
        pltpu.sync_copy(zeros_hbm, acc_sh.at[sl])
        if spmem_table:
            tbl = refs[13]
            pltpu.sync_copy(table_hbm.at[sl], tbl.at[sl])
        else:
            tbl = table_hbm
        plsc.subcore_barrier()

        base = (c * NS + s) * per_tile
        h = chunk // 2

        def off(kk):
            return pl.ds(base + kk * chunk, chunk)

        def idx_start(kk, b):
            pltpu.async_copy(dst_hbm.at[off(kk)], didx[b], isem[b])
            pltpu.async_copy(src_hbm.at[off(kk)], sidx[b], isem[b])

        def idx_wait(kk, b):
            pltpu.make_async_copy(dst_hbm.at[off(kk)], didx[b], isem[b]).wait()
            pltpu.make_async_copy(src_hbm.at[off(kk)], sidx[b],
                                  isem[b]).wait()

        def gather_start(b):
            pltpu.async_copy(tbl.at[sidx[b].at[pl.ds(0, h)]],
                             rows[b].at[pl.ds(0, h)], gsem[b])
            pltpu.async_copy(tbl.at[sidx[b].at[pl.ds(h, h)]],
                             rows[b].at[pl.ds(h, h)], gsem2[b])

        def gather_wait(b):
            pltpu.make_async_copy(tbl.at[sidx[b].at[pl.ds(0, h)]],
                                  rows[b].at[pl.ds(0, h)], gsem[b]).wait()
            pltpu.make_async_copy(tbl.at[sidx[b].at[pl.ds(h, h)]],
                                  rows[b].at[pl.ds(h, h)], gsem2[b]).wait()

        idx_start(0, 0)
        idx_wait(0, 0)
        gather_start(0)
        if n_chunks > 1:
            idx_start(1, 1)
        for kk in range(n_chunks):
            b = kk & 1
            nb = 1 - b
            gather_wait(b)
            if kk + 1 < n_chunks:
                idx_wait(kk + 1, nb)
                gather_start(nb)
            pltpu.sync_copy(rows[b], acc_sh.at[didx[b]], add=True)
            if kk + 2 < n_chunks:
                idx_start(kk + 2, b)

        plsc.subcore_barrier()
        pltpu.sync_copy(acc_sh.at[sl], out_hbm.at[c].at[sl])

    return k


def _deg_kernel(npad, e_pad):
    F = 8
    per_tile = e_pad // NT
    n_chunks = per_tile // CHUNK
    slice_sz = npad // NS

    scratch = (
        [pltpu.VMEM((CHUNK,), jnp.int32) for _ in range(2)] +
        [pltpu.VMEM((CHUNK, F), jnp.float32),
         pltpu.VMEM_SHARED((npad, F), jnp.float32),
         pltpu.SemaphoreType.DMA, pltpu.SemaphoreType.DMA]
    )

    @functools.partial(
        pl.kernel,
        out_type=jax.ShapeDtypeStruct((NC, npad, F), jnp.float32),
        mesh=plsc.VectorSubcoreMesh(**_MESH),
        scratch_types=scratch,
        compiler_params=pltpu.CompilerParams(use_tc_tiling_on_sc=False),
    )
    def k(dst_hbm, ones_hbm, zeros_hbm, out_hbm, *refs):
        didx = refs[0:2]
        ones_v = refs[2]
        acc_sh = refs[3]
        isem = refs[4:6]
        c = lax.axis_index("c")
        s = lax.axis_index("s")
        sl = pl.ds(s * slice_sz, slice_sz)
        pltpu.sync_copy(zeros_hbm, acc_sh.at[sl])
        pltpu.sync_copy(ones_hbm, ones_v)
        plsc.subcore_barrier()
        base = (c * NS + s) * per_tile

        def off(kk):
            return pl.ds(base + kk * CHUNK, CHUNK)

        def idx_start(kk, b):
            pltpu.async_copy(dst_hbm.at[off(kk)], didx[b], isem[b])

        def idx_wait(kk, b):
            pltpu.make_async_copy(dst_hbm.at[off(kk)], didx[b], isem[b]).wait()

        idx_start(0, 0)
        if n_chunks > 1:
            idx_start(1, 1)
        for kk in range(n_chunks):
            b = kk & 1
            idx_wait(kk, b)
            pltpu.sync_copy(ones_v, acc_sh.at[didx[b]], add=True)
            if kk + 2 < n_chunks:
                idx_start(kk + 2, b)

        plsc.subcore_barrier()
        pltpu.sync_copy(acc_sh.at[sl], out_hbm.at[c].at[sl])

    return k


_SPREAD = np.zeros((32, 128), np.float32)
for _j in range(16):
    for _k in range(2):
        _SPREAD[2 * _j + _k, 8 * _j + _k] = 1.0


def _prep_call(npad, deg_p, x32):
    def body(degp, xr, spread, disf, xnf):
        deg = degp[0] + degp[1] + 1.0
        d = lax.rsqrt(deg)
        disf[...] = d
        xf = jnp.dot(xr[...], spread[...], preferred_element_type=jnp.float32)
        xnf[...] = xf * d

    G = npad // 16
    R = 784
    g = G // R
    return pl.pallas_call(
        body,
        grid=(g,),
        in_specs=[
            pl.BlockSpec((NC, R, 128), lambda i: (0, i, 0)),
            pl.BlockSpec((R, 32), lambda i: (i, 0)),
            pl.BlockSpec((32, 128), lambda i: (0, 0)),
        ],
        out_specs=(
            pl.BlockSpec((R, 128), lambda i: (i, 0)),
            pl.BlockSpec((R, 128), lambda i: (i, 0)),
        ),
        out_shape=(
            jax.ShapeDtypeStruct((G, 128), jnp.float32),
            jax.ShapeDtypeStruct((G, 128), jnp.float32),
        ),
    )(deg_p, x32, _SPREAD)


def _mid_call(npad, t1_p, xnf, disf, W1big, b1big, W2big):
    G = npad // 16
    R = 784
    g = G // R

    def body(tp, xnb, disb, w1, bb1, w2, gnf):
        agg = (tp[0] + tp[1] + xnb[...]) * disb[...]
        h1 = jnp.dot(agg, w1[...], preferred_element_type=jnp.float32)
        h1 = jnp.maximum(h1 + bb1[...], 0.0)
        gnf[...] = jnp.dot(h1, w2[...],
                           preferred_element_type=jnp.float32) * disb[...]

    return pl.pallas_call(
        body,
        grid=(g,),
        in_specs=[
            pl.BlockSpec((NC, R, 128), lambda i: (0, i, 0)),
            pl.BlockSpec((R, 128), lambda i: (i, 0)),
            pl.BlockSpec((R, 128), lambda i: (i, 0)),
            pl.BlockSpec((128, 256), lambda i: (0, 0)),
            pl.BlockSpec((1, 256), lambda i: (0, 0)),
            pl.BlockSpec((256, 128), lambda i: (0, 0)),
        ],
        out_specs=pl.BlockSpec((R, 128), lambda i: (i, 0)),
        out_shape=jax.ShapeDtypeStruct((G, 128), jnp.float32),
    )(t1_p, xnf, disf, W1big, b1big.reshape(1, 256), W2big)


def _post_call(npad, t2_p, gnf, disf):
    G = npad // 16
    R = 784
    g = G // R

    def body(tp, gnb, disb, aggf):
        aggf[...] = (tp[0] + tp[1] + gnb[...]) * disb[...]

    return pl.pallas_call(
        body,
        grid=(g,),
        in_specs=[
            pl.BlockSpec((NC, R, 128), lambda i: (0, i, 0)),
            pl.BlockSpec((R, 128), lambda i: (i, 0)),
            pl.BlockSpec((R, 128), lambda i: (i, 0)),
        ],
        out_specs=pl.BlockSpec((R, 128), lambda i: (i, 0)),
        out_shape=jax.ShapeDtypeStruct((G, 128), jnp.float32),
    )(t2_p, gnf, disf)


def _final_call(n, agg2, fcW, b2, fcb):
    R = 4000
    g = n // R

    def body(ab, w, bb2, bfc, out):
        cvec = jnp.dot(bb2[...], w[...],
                       preferred_element_type=jnp.float32) + bfc[...]
        out[...] = jnp.dot(ab[...], w[...],
                           preferred_element_type=jnp.float32) + cvec

    return pl.pallas_call(
        body,
        grid=(g,),
        in_specs=[
            pl.BlockSpec((R, 8), lambda i: (i, 0)),
            pl.BlockSpec((8, 256), lambda i: (0, 0)),
            pl.BlockSpec((1, 8), lambda i: (0, 0)),
            pl.BlockSpec((1, 256), lambda i: (0, 0)),
        ],
        out_specs=pl.BlockSpec((R, 256), lambda i: (i, 0)),
        out_shape=jax.ShapeDtypeStruct((n, 256), jnp.float32),
    )(agg2, fcW, b2.reshape(1, 8), fcb.reshape(1, 256))


@jax.jit
def kernel(x, edge_index, W1, b1, W2, b2, fcW, fcb):
    n = x.shape[0]
    e = edge_index.shape[1]
    npad = _round_up(n + 1, 2048)
    e_pad = _round_up(e, NT * CHUNK)

    src = edge_index[0]
    dst = edge_index[1]
    if e_pad != e:
        pad = jnp.full((e_pad - e,), n, dtype=jnp.int32)
        src = jnp.concatenate([src, pad])
        dst = jnp.concatenate([dst, pad])

    G = npad // 16
    W1p = jnp.zeros((8, 16), jnp.float32).at[:2].set(W1)
    W1big = jnp.kron(jnp.eye(16, dtype=jnp.float32), W1p)
    W2big = jnp.kron(jnp.eye(16, dtype=jnp.float32), W2)
    b1big = jnp.tile(b1, 16)
    zeros_s = np.zeros((npad // NS, 8), np.float32)
    ones_c = np.ones((CHUNK, 8), np.float32)
    x32 = jnp.pad(x.reshape(n * 2), (0, (npad - n) * 2)).reshape(G, 32)

    deg_p = _deg_kernel(npad, e_pad)(dst, ones_c, zeros_s)

    disf, xnf = _prep_call(npad, deg_p.reshape(NC, G, 128), x32)

    t1_p = _gs_kernel(npad, e_pad)(src, dst, xnf.reshape(npad, 8), zeros_s)

    gnf = _mid_call(npad, t1_p.reshape(NC, G, 128), xnf, disf,
                    W1big, b1big, W2big)

    t2_p = _gs_kernel(npad, e_pad)(src, dst, gnf.reshape(npad, 8), zeros_s)

    agg2f = _post_call(npad, t2_p.reshape(NC, G, 128), gnf, disf)
    return _final_call(n, agg2f.reshape(npad, 8), fcW, b2, fcb)
